# baseline (speedup 1.0000x reference)
import functools

import jax
import jax.numpy as jnp
from jax import lax
from jax.experimental import pallas as pl
from jax.experimental.pallas import tpu as pltpu
from jax.experimental.pallas import tpu_sc as plsc

_R = 10
_MAXR = 2 * _R - 1
_KBLK = 256


def _interact_kernel(xf_ref, xb_ref, w_ref, b_ref, xw_ref, xwt_ref, s_ref):
    k = pl.program_id(0)
    x = xf_ref[...]
    w = w_ref[...]

    @pl.when(k == 0)
    def _init():
        bvec = b_ref[...]
        xw_ref[...] = jnp.broadcast_to(bvec, xw_ref.shape)
        s_ref[...] = jnp.sum(x * bvec, axis=-1, keepdims=True)

    xk = xb_ref[...]
    p = jnp.dot(xk, w, preferred_element_type=jnp.float32)
    q = jax.lax.dot_general(x, w, (((1,), (1,)), ((), ())),
                            preferred_element_type=jnp.float32)
    xw_ref[...] += p
    xwt_ref[...] = q
    s_ref[...] += jnp.sum(p * x, axis=-1, keepdims=True)


def _interact(u, W, b2):
    B, D = u.shape
    grid = (D // _KBLK,)
    return pl.pallas_call(
        _interact_kernel,
        grid=grid,
        in_specs=[
            pl.BlockSpec((B, D), lambda k: (0, 0)),
            pl.BlockSpec((B, _KBLK), lambda k: (0, k)),
            pl.BlockSpec((_KBLK, D), lambda k: (k, 0)),
            pl.BlockSpec((1, D), lambda k: (0, 0)),
        ],
        out_specs=[
            pl.BlockSpec((B, D), lambda k: (0, 0)),
            pl.BlockSpec((B, _KBLK), lambda k: (0, k)),
            pl.BlockSpec((B, 1), lambda k: (0, 0)),
        ],
        out_shape=[
            jax.ShapeDtypeStruct((B, D), jnp.float32),
            jax.ShapeDtypeStruct((B, D), jnp.float32),
            jax.ShapeDtypeStruct((B, 1), jnp.float32),
        ],
    )(u, u, W, b2)


def _sc_sampler_build(B, D):
    info = plsc.get_sparse_core_info()
    NC, NS, L = info.num_cores, info.num_subcores, info.num_lanes
    NW = NC * NS
    RPW = B // NW
    NCHUNK = D // L

    mesh = plsc.VectorSubcoreMesh(core_axis_name="c", subcore_axis_name="s")

    @functools.partial(
        pl.kernel, mesh=mesh,
        compiler_params=pltpu.CompilerParams(needs_layout_passes=False),
        out_type=[
            jax.ShapeDtypeStruct((B, D), jnp.float32),
            jax.ShapeDtypeStruct((NW, _MAXR, L), jnp.int32),
            jax.ShapeDtypeStruct((NW, _MAXR, L), jnp.float32),
            jax.ShapeDtypeStruct((NW, _MAXR, L), jnp.float32),
            jax.ShapeDtypeStruct((NW, L), jnp.float32),
        ],
        scratch_types=[
            pltpu.VMEM((B // (NC * NS), D), jnp.float32),
            pltpu.VMEM((B // (NC * NS), D), jnp.float32),
            pltpu.VMEM((B // (NC * NS), D), jnp.float32),
            pltpu.VMEM((B // (NC * NS), D), jnp.float32),
            pltpu.VMEM((B // (NC * NS), D), jnp.float32),
            pltpu.VMEM((_MAXR, 16), jnp.int32),
            pltpu.VMEM((_MAXR, 16), jnp.float32),
            pltpu.VMEM((_MAXR, 16), jnp.float32),
            pltpu.VMEM((16,), jnp.float32),
            pltpu.VMEM((16,), jnp.int32),
        ],
    )
    def samp(xw_h, xwt_h, x_h, g_h, rad_h,
             y_h, idx_h, sel_h, s_h, m_h,
             xw_v, xwt_v, x_v, l_v, g_v, idx_st, sel_st, s_st, m_st, rad_v):
        wid = lax.axis_index("s") * NC + lax.axis_index("c")
        base = wid * RPW
        pltpu.sync_copy(xw_h.at[pl.ds(base, RPW)], xw_v)
        pltpu.sync_copy(xwt_h.at[pl.ds(base, RPW)], xwt_v)
        pltpu.sync_copy(x_h.at[pl.ds(base, RPW)], x_v)
        pltpu.sync_copy(rad_h.at[wid], rad_v)
        rad = rad_v[...]
        lane = lax.iota(jnp.int32, L)

        m_vec = jnp.zeros((L,), jnp.float32)
        s_vec = jnp.zeros((L,), jnp.float32)
        for r in range(RPW):
            def _initb(c, vmax, r=r):
                sl = pl.ds(c * L, L)
                lch = (1.0 - 2.0 * x_v[r, sl]) * (xw_v[r, sl] + xwt_v[r, sl]) * 0.5
                l_v[r, sl] = lch
                return jnp.maximum(vmax, jnp.abs(lch))
            vmax = lax.fori_loop(0, NCHUNK, _initb,
                                 jnp.zeros((L,), jnp.float32))
            m_r = jnp.max(vmax)

            def _sumb(c, acc, r=r, m_r=m_r):
                return acc + jnp.exp(l_v[r, pl.ds(c * L, L)] - m_r)
            accv = lax.fori_loop(0, NCHUNK, _sumb, jnp.zeros((L,), jnp.float32))
            lm = lane == r
            m_vec = jnp.where(lm, m_r, m_vec)
            s_vec = jnp.where(lm, jnp.sum(accv), s_vec)
        m_st[...] = m_vec
        pltpu.sync_copy(m_st, m_h.at[wid])

        def _step(s, s_vec):
            pltpu.sync_copy(g_h.at[s, pl.ds(base, RPW)], g_v)
            flipm_all = jnp.broadcast_to(s, (L,)) < rad
            s_pre = s_vec
            idx_all = jnp.zeros((L,), jnp.int32)
            sel_all = jnp.zeros((L,), jnp.float32)
            for r in range(RPW):
                def _amax(c, carry, r=r):
                    vmax, vc = carry
                    sl = pl.ds(c * L, L)
                    t = l_v[r, sl] + g_v[r, sl]
                    gt = t > vmax
                    return (jnp.where(gt, t, vmax), jnp.where(gt, c, vc))
                vmax, vc = lax.fori_loop(
                    0, NCHUNK, _amax,
                    (jnp.full((L,), -jnp.inf, jnp.float32),
                     jnp.zeros((L,), jnp.int32)))
                gmax = jnp.max(vmax)
                dcand = jnp.where(vmax == gmax, vc * L + lane, jnp.int32(2**30))
                idx_r = jnp.min(dcand)
                idxv = jnp.broadcast_to(idx_r, (L,))
                rv = jnp.broadcast_to(jnp.int32(r), (L,))
                selv = plsc.load_gather(l_v, [rv, idxv])
                lm = lane == r
                flip_r = flipm_all & lm
                idx_all = jnp.where(lm, idx_r, idx_all)
                sel_all = jnp.where(lm, selv, sel_all)
                plsc.store_scatter(l_v, [rv, idxv], -selv, mask=flip_r)
                bitv = plsc.load_gather(x_v, [rv, idxv])
                plsc.store_scatter(x_v, [rv, idxv], 1.0 - bitv, mask=flip_r)
                ds = jnp.exp(-selv - m_vec) - jnp.exp(selv - m_vec)
                s_vec = s_vec + jnp.where(flip_r, ds, 0.0)
            sall = jnp.broadcast_to(s, (L,))
            plsc.store_scatter(idx_st, [sall, lane], idx_all)
            plsc.store_scatter(sel_st, [sall, lane], sel_all)
            plsc.store_scatter(s_st, [sall, lane], s_pre)
            return s_vec

        lax.fori_loop(0, _MAXR, _step, s_vec)
        pltpu.sync_copy(x_v, y_h.at[pl.ds(base, RPW)])
        pltpu.sync_copy(idx_st, idx_h.at[wid])
        pltpu.sync_copy(sel_st, sel_h.at[wid])
        pltpu.sync_copy(s_st, s_h.at[wid])

    return samp, NW, RPW, L


def _backward_kernel(yw_ref, ywt_ref, y_ref, x_ref, idx_ref, rm_ref,
                     sy_ref, sx_ref, self_ref, sf_ref, mf_ref, u_ref,
                     out_ref):
    lse_fwd = mf_ref[...] + jnp.log(sf_ref[...])
    lf = sx_ref[...] + jnp.sum(rm_ref[...] * (self_ref[...] - lse_fwd),
                               axis=-1, keepdims=True)
    y = y_ref[...]
    delta = 1.0 - 2.0 * y
    grad = yw_ref[...] + ywt_ref[...]
    l = delta * grad * 0.5
    m = jnp.max(jnp.abs(l), axis=-1, keepdims=True)
    S = jnp.sum(jnp.exp(l - m), axis=-1, keepdims=True)
    iota = jax.lax.broadcasted_iota(jnp.int32, l.shape, 1)
    acc = jnp.zeros_like(m)
    for s in range(_MAXR - 1, -1, -1):
        idx = idx_ref[s]
        onehot = iota == idx
        sel = jnp.sum(jnp.where(onehot, l, 0.0), axis=-1, keepdims=True)
        mask = rm_ref[:, s:s + 1]
        acc += mask * (sel - (m + jnp.log(S)))
        if s > 0:
            do = onehot & (mask > 0.0)
            l = jnp.where(do, -l, l)
            S = S + mask * (jnp.exp(-sel - m) - jnp.exp(sel - m))
    log_backwd = acc + sy_ref[...]
    log_acc = log_backwd - lf
    accept = (jnp.exp(log_acc) >= u_ref[...]).astype(jnp.float32)
    out_ref[...] = y * accept + (1.0 - accept) * x_ref[...]


def _backward(yw, ywt, y, x, idxarr, rmask, sy, sx, selfwd, sfwd, mfwd, u):
    B, D = x.shape
    return pl.pallas_call(
        _backward_kernel,
        out_shape=jax.ShapeDtypeStruct((B, D), jnp.float32),
    )(yw, ywt, y, x, idxarr, rmask, sy, sx, selfwd, sfwd, mfwd, u)


def kernel(x, W, b):
    B, D = x.shape
    key = jax.random.key(42)
    k_r, k_loop, k_acc = jax.random.split(key, 3)
    radius = jax.random.randint(k_r, (B, 1), 1, 2 * _R)
    r_mask = (jnp.arange(_MAXR)[None, :] < radius).astype(jnp.float32)
    G = jax.vmap(lambda s: jax.random.gumbel(
        jax.random.fold_in(k_loop, s), (B, D), jnp.float32))(jnp.arange(_MAXR))
    u = jax.random.uniform(k_acc, (B,)).reshape(B, 1)
    b2 = b.reshape(1, D)

    samp, NW, RPW, L = _sc_sampler_build(B, D)
    rad32 = jnp.zeros((NW, L), jnp.int32).at[:, :RPW].set(
        radius.reshape(NW, RPW))

    xw, xwt, sx = _interact(x, W, b2)
    y, idxw, selw, sw, mw = samp(xw, xwt, x, G, rad32)
    idx_arr = idxw[:, :, :RPW].transpose(0, 2, 1).reshape(B, _MAXR)
    selfwd = selw[:, :, :RPW].transpose(0, 2, 1).reshape(B, _MAXR)
    sfwd = sw[:, :, :RPW].transpose(0, 2, 1).reshape(B, _MAXR)
    mfwd = mw[:, :RPW].reshape(B, 1)
    idx_sb = idx_arr.T.reshape(_MAXR, B, 1)

    yw, ywt, sy = _interact(y, W, b2)
    return _backward(yw, ywt, y, x, idx_sb, r_mask, sy, sx,
                     selfwd, sfwd, mfwd, u)


# speedup vs baseline: 1.2204x; 1.2204x over previous
import functools

import jax
import jax.numpy as jnp
from jax import lax
from jax.experimental import pallas as pl
from jax.experimental.pallas import tpu as pltpu
from jax.experimental.pallas import tpu_sc as plsc

_R = 10
_MAXR = 2 * _R - 1
_KBLK = 256


def _interact_kernel(xf_ref, xb_ref, w_ref, b_ref, xw_ref, xwt_ref, s_ref):
    k = pl.program_id(0)
    x = xf_ref[...]
    w = w_ref[...]

    @pl.when(k == 0)
    def _init():
        bvec = b_ref[...]
        xw_ref[...] = jnp.broadcast_to(bvec, xw_ref.shape)
        s_ref[...] = jnp.sum(x * bvec, axis=-1, keepdims=True)

    xk = xb_ref[...]
    p = jnp.dot(xk, w, preferred_element_type=jnp.float32)
    q = jax.lax.dot_general(x, w, (((1,), (1,)), ((), ())),
                            preferred_element_type=jnp.float32)
    xw_ref[...] += p
    xwt_ref[...] = q
    s_ref[...] += jnp.sum(p * x, axis=-1, keepdims=True)


def _interact(u, W, b2):
    B, D = u.shape
    grid = (D // _KBLK,)
    return pl.pallas_call(
        _interact_kernel,
        grid=grid,
        in_specs=[
            pl.BlockSpec((B, D), lambda k: (0, 0)),
            pl.BlockSpec((B, _KBLK), lambda k: (0, k)),
            pl.BlockSpec((_KBLK, D), lambda k: (k, 0)),
            pl.BlockSpec((1, D), lambda k: (0, 0)),
        ],
        out_specs=[
            pl.BlockSpec((B, D), lambda k: (0, 0)),
            pl.BlockSpec((B, _KBLK), lambda k: (0, k)),
            pl.BlockSpec((B, 1), lambda k: (0, 0)),
        ],
        out_shape=[
            jax.ShapeDtypeStruct((B, D), jnp.float32),
            jax.ShapeDtypeStruct((B, D), jnp.float32),
            jax.ShapeDtypeStruct((B, 1), jnp.float32),
        ],
    )(u, u, W, b2)


def _sc_sampler_build(B, D):
    info = plsc.get_sparse_core_info()
    NC, NS, L = info.num_cores, info.num_subcores, info.num_lanes
    NW = NC * NS
    RPW = B // NW
    NCHUNK = D // L

    mesh = plsc.VectorSubcoreMesh(core_axis_name="c", subcore_axis_name="s")

    @functools.partial(
        pl.kernel, mesh=mesh,
        compiler_params=pltpu.CompilerParams(needs_layout_passes=False),
        out_type=[
            jax.ShapeDtypeStruct((B, D), jnp.float32),
            jax.ShapeDtypeStruct((NW, _MAXR, L), jnp.int32),
            jax.ShapeDtypeStruct((NW, _MAXR, L), jnp.float32),
            jax.ShapeDtypeStruct((NW, _MAXR, L), jnp.float32),
            jax.ShapeDtypeStruct((NW, L), jnp.float32),
        ],
        scratch_types=[
            pltpu.VMEM((B // (NC * NS), D), jnp.float32),
            pltpu.VMEM((B // (NC * NS), D), jnp.float32),
            pltpu.VMEM((B // (NC * NS), D), jnp.float32),
            pltpu.VMEM((B // (NC * NS), D), jnp.float32),
            pltpu.VMEM((B // (NC * NS), D), jnp.float32),
            pltpu.VMEM((B // (NC * NS), D), jnp.float32),
            pltpu.VMEM((_MAXR, 16), jnp.int32),
            pltpu.VMEM((_MAXR, 16), jnp.float32),
            pltpu.VMEM((_MAXR, 16), jnp.float32),
            pltpu.VMEM((16,), jnp.float32),
            pltpu.VMEM((16,), jnp.int32),
            pltpu.SemaphoreType.DMA,
            pltpu.SemaphoreType.DMA,
        ],
    )
    def samp(xw_h, xwt_h, x_h, g_h, rad_h,
             y_h, idx_h, sel_h, s_h, m_h,
             xw_v, xwt_v, x_v, l_v, g0_v, g1_v,
             idx_st, sel_st, s_st, m_st, rad_v, sem0, sem1):
        wid = lax.axis_index("s") * NC + lax.axis_index("c")
        base = wid * RPW
        gbufs = (g0_v, g1_v)
        sems = (sem0, sem1)
        pend = pltpu.async_copy(g_h.at[0, pl.ds(base, RPW)], g0_v, sem0)
        pltpu.sync_copy(xw_h.at[pl.ds(base, RPW)], xw_v)
        pltpu.sync_copy(xwt_h.at[pl.ds(base, RPW)], xwt_v)
        pltpu.sync_copy(x_h.at[pl.ds(base, RPW)], x_v)
        pltpu.sync_copy(rad_h.at[wid], rad_v)
        rad = rad_v[...]
        lane = lax.iota(jnp.int32, L)
        _U = 2

        def _initb(c, carry):
            out = list(carry)
            for u in range(_U):
                for r in range(RPW):
                    sl = pl.ds((c * _U + u) * L, L)
                    lch = ((1.0 - 2.0 * x_v[r, sl])
                           * (xw_v[r, sl] + xwt_v[r, sl]) * 0.5)
                    l_v[r, sl] = lch
                    out[r] = jnp.maximum(out[r], jnp.abs(lch))
            return tuple(out)
        vmaxes = lax.fori_loop(0, NCHUNK // _U, _initb,
                               (jnp.zeros((L,), jnp.float32),) * RPW)
        m_rs = [jnp.max(vmaxes[r]) for r in range(RPW)]

        def _sumb(c, carry):
            out = list(carry)
            for u in range(_U):
                for r in range(RPW):
                    sl = pl.ds((c * _U + u) * L, L)
                    out[r] = out[r] + jnp.exp(l_v[r, sl] - m_rs[r])
            return tuple(out)
        accs = lax.fori_loop(0, NCHUNK // _U, _sumb,
                             (jnp.zeros((L,), jnp.float32),) * RPW)
        m_vec = jnp.zeros((L,), jnp.float32)
        s_vec = jnp.zeros((L,), jnp.float32)
        for r in range(RPW):
            lm = lane == r
            m_vec = jnp.where(lm, m_rs[r], m_vec)
            s_vec = jnp.where(lm, jnp.sum(accs[r]), s_vec)
        m_st[...] = m_vec
        pltpu.sync_copy(m_st, m_h.at[wid])

        neg_inf = jnp.full((L,), -jnp.inf, jnp.float32)
        zero_i = jnp.zeros((L,), jnp.int32)
        for s in range(_MAXR):
            g_v = gbufs[s % 2]
            if s + 1 < _MAXR:
                nxt = pltpu.async_copy(
                    g_h.at[s + 1, pl.ds(base, RPW)],
                    gbufs[(s + 1) % 2], sems[(s + 1) % 2])
            pend.wait()
            pend = nxt if s + 1 < _MAXR else None
            flipm_all = jnp.broadcast_to(jnp.int32(s), (L,)) < rad
            s_pre = s_vec
            idx_all = jnp.zeros((L,), jnp.int32)
            sel_all = jnp.zeros((L,), jnp.float32)

            def _amax(c, carry, g_v=g_v):
                out = list(carry)
                for u in range(_U):
                    for r in range(RPW):
                        ci = c * _U + u
                        sl = pl.ds(ci * L, L)
                        t = l_v[r, sl] + g_v[r, sl]
                        vmax, vc = out[2 * r], out[2 * r + 1]
                        gt = t > vmax
                        out[2 * r] = jnp.where(gt, t, vmax)
                        out[2 * r + 1] = jnp.where(gt, ci, vc)
                return tuple(out)
            carry = lax.fori_loop(0, NCHUNK // _U, _amax,
                                  (neg_inf, zero_i) * RPW)
            for r in range(RPW):
                vmax, vc = carry[2 * r], carry[2 * r + 1]
                gmax = jnp.max(vmax)
                dcand = jnp.where(vmax == gmax, vc * L + lane, jnp.int32(2**30))
                idx_r = jnp.min(dcand)
                idxv = jnp.broadcast_to(idx_r, (L,))
                rv = jnp.broadcast_to(jnp.int32(r), (L,))
                selv = plsc.load_gather(l_v, [rv, idxv])
                lm = lane == r
                flip_r = flipm_all & lm
                idx_all = jnp.where(lm, idx_r, idx_all)
                sel_all = jnp.where(lm, selv, sel_all)
                plsc.store_scatter(l_v, [rv, idxv], -selv, mask=flip_r)
                bitv = plsc.load_gather(x_v, [rv, idxv])
                plsc.store_scatter(x_v, [rv, idxv], 1.0 - bitv, mask=flip_r)
                ds = jnp.exp(-selv - m_vec) - jnp.exp(selv - m_vec)
                s_vec = s_vec + jnp.where(flip_r, ds, 0.0)
            sall = jnp.broadcast_to(jnp.int32(s), (L,))
            plsc.store_scatter(idx_st, [sall, lane], idx_all)
            plsc.store_scatter(sel_st, [sall, lane], sel_all)
            plsc.store_scatter(s_st, [sall, lane], s_pre)
        pltpu.sync_copy(x_v, y_h.at[pl.ds(base, RPW)])
        pltpu.sync_copy(idx_st, idx_h.at[wid])
        pltpu.sync_copy(sel_st, sel_h.at[wid])
        pltpu.sync_copy(s_st, s_h.at[wid])

    return samp, NW, RPW, L


def _backward_kernel(yw_ref, ywt_ref, y_ref, x_ref, idx_ref, rm_ref,
                     sy_ref, sx_ref, self_ref, sf_ref, mf_ref, u_ref,
                     out_ref):
    lse_fwd = mf_ref[...] + jnp.log(sf_ref[...])
    lf = sx_ref[...] + jnp.sum(rm_ref[...] * (self_ref[...] - lse_fwd),
                               axis=-1, keepdims=True)
    y = y_ref[...]
    delta = 1.0 - 2.0 * y
    grad = yw_ref[...] + ywt_ref[...]
    l = delta * grad * 0.5
    m = jnp.max(jnp.abs(l), axis=-1, keepdims=True)
    S = jnp.sum(jnp.exp(l - m), axis=-1, keepdims=True)
    iota = jax.lax.broadcasted_iota(jnp.int32, l.shape, 1)
    acc = jnp.zeros_like(m)
    for s in range(_MAXR - 1, -1, -1):
        idx = idx_ref[s]
        onehot = iota == idx
        sel = jnp.sum(jnp.where(onehot, l, 0.0), axis=-1, keepdims=True)
        mask = rm_ref[:, s:s + 1]
        acc += mask * (sel - (m + jnp.log(S)))
        if s > 0:
            do = onehot & (mask > 0.0)
            l = jnp.where(do, -l, l)
            S = S + mask * (jnp.exp(-sel - m) - jnp.exp(sel - m))
    log_backwd = acc + sy_ref[...]
    log_acc = log_backwd - lf
    accept = (jnp.exp(log_acc) >= u_ref[...]).astype(jnp.float32)
    out_ref[...] = y * accept + (1.0 - accept) * x_ref[...]


def _backward(yw, ywt, y, x, idxarr, rmask, sy, sx, selfwd, sfwd, mfwd, u):
    B, D = x.shape
    return pl.pallas_call(
        _backward_kernel,
        out_shape=jax.ShapeDtypeStruct((B, D), jnp.float32),
    )(yw, ywt, y, x, idxarr, rmask, sy, sx, selfwd, sfwd, mfwd, u)


def kernel(x, W, b):
    B, D = x.shape
    key = jax.random.key(42)
    k_r, k_loop, k_acc = jax.random.split(key, 3)
    radius = jax.random.randint(k_r, (B, 1), 1, 2 * _R)
    r_mask = (jnp.arange(_MAXR)[None, :] < radius).astype(jnp.float32)
    G = jax.vmap(lambda s: jax.random.gumbel(
        jax.random.fold_in(k_loop, s), (B, D), jnp.float32))(jnp.arange(_MAXR))
    u = jax.random.uniform(k_acc, (B,)).reshape(B, 1)
    b2 = b.reshape(1, D)

    samp, NW, RPW, L = _sc_sampler_build(B, D)
    rad32 = jnp.zeros((NW, L), jnp.int32).at[:, :RPW].set(
        radius.reshape(NW, RPW))

    xw, xwt, sx = _interact(x, W, b2)
    y, idxw, selw, sw, mw = samp(xw, xwt, x, G, rad32)
    idx_arr = idxw[:, :, :RPW].transpose(0, 2, 1).reshape(B, _MAXR)
    selfwd = selw[:, :, :RPW].transpose(0, 2, 1).reshape(B, _MAXR)
    sfwd = sw[:, :, :RPW].transpose(0, 2, 1).reshape(B, _MAXR)
    mfwd = mw[:, :RPW].reshape(B, 1)
    idx_sb = idx_arr.T.reshape(_MAXR, B, 1)

    yw, ywt, sy = _interact(y, W, b2)
    return _backward(yw, ywt, y, x, idx_sb, r_mask, sy, sx,
                     selfwd, sfwd, mfwd, u)


# speedup vs baseline: 1.2557x; 1.0289x over previous
import functools

import jax
import jax.numpy as jnp
from jax import lax
from jax.experimental import pallas as pl
from jax.experimental.pallas import tpu as pltpu
from jax.experimental.pallas import tpu_sc as plsc

_R = 10
_MAXR = 2 * _R - 1
_KBLK = 256


def _interact_kernel(xf_ref, xb_ref, w_ref, b_ref, xw_ref, xwt_ref, s_ref):
    k = pl.program_id(0)
    x = xf_ref[...]
    w = w_ref[...]

    @pl.when(k == 0)
    def _init():
        bvec = b_ref[...]
        xw_ref[...] = jnp.broadcast_to(bvec, xw_ref.shape)
        s_ref[...] = jnp.sum(x * bvec, axis=-1, keepdims=True)

    xk = xb_ref[...]
    p = jnp.dot(xk, w, preferred_element_type=jnp.float32)
    q = jax.lax.dot_general(x, w, (((1,), (1,)), ((), ())),
                            preferred_element_type=jnp.float32)
    xw_ref[...] += p
    xwt_ref[...] = q
    s_ref[...] += jnp.sum(p * x, axis=-1, keepdims=True)


def _interact(u, W, b2):
    B, D = u.shape
    grid = (D // _KBLK,)
    return pl.pallas_call(
        _interact_kernel,
        grid=grid,
        in_specs=[
            pl.BlockSpec((B, D), lambda k: (0, 0)),
            pl.BlockSpec((B, _KBLK), lambda k: (0, k)),
            pl.BlockSpec((_KBLK, D), lambda k: (k, 0)),
            pl.BlockSpec((1, D), lambda k: (0, 0)),
        ],
        out_specs=[
            pl.BlockSpec((B, D), lambda k: (0, 0)),
            pl.BlockSpec((B, _KBLK), lambda k: (0, k)),
            pl.BlockSpec((B, 1), lambda k: (0, 0)),
        ],
        out_shape=[
            jax.ShapeDtypeStruct((B, D), jnp.float32),
            jax.ShapeDtypeStruct((B, D), jnp.float32),
            jax.ShapeDtypeStruct((B, 1), jnp.float32),
        ],
    )(u, u, W, b2)


def _sc_sampler_build(B, D):
    info = plsc.get_sparse_core_info()
    NC, NS, L = info.num_cores, info.num_subcores, info.num_lanes
    NW = NC * NS
    RPW = B // NW
    NCHUNK = D // L

    mesh = plsc.VectorSubcoreMesh(core_axis_name="c", subcore_axis_name="s")

    @functools.partial(
        pl.kernel, mesh=mesh,
        compiler_params=pltpu.CompilerParams(needs_layout_passes=False),
        out_type=[
            jax.ShapeDtypeStruct((B, D), jnp.float32),
            jax.ShapeDtypeStruct((B, _MAXR), jnp.int32),
            jax.ShapeDtypeStruct((B, _MAXR), jnp.float32),
            jax.ShapeDtypeStruct((B, _MAXR), jnp.float32),
        ],
        scratch_types=[
            pltpu.VMEM((B // (NC * NS), D), jnp.float32),
            pltpu.VMEM((B // (NC * NS), D), jnp.float32),
            pltpu.VMEM((B // (NC * NS), D), jnp.float32),
            pltpu.VMEM((B // (NC * NS), D), jnp.float32),
            pltpu.VMEM((B // (NC * NS), D), jnp.float32),
            pltpu.VMEM((B // (NC * NS), D), jnp.float32),
            pltpu.VMEM((16, _MAXR), jnp.int32),
            pltpu.VMEM((16, _MAXR), jnp.float32),
            pltpu.VMEM((16, _MAXR), jnp.float32),
            pltpu.VMEM((16,), jnp.int32),
            pltpu.SemaphoreType.DMA,
            pltpu.SemaphoreType.DMA,
        ],
    )
    def samp(xw_h, xwt_h, x_h, g_h, rad_h,
             y_h, idx_h, sel_h, s_h,
             xw_v, xwt_v, x_v, l_v, g0_v, g1_v,
             idx_st, sel_st, s_st, rad_v, sem0, sem1):
        wid = lax.axis_index("s") * NC + lax.axis_index("c")
        base = wid * RPW
        gbufs = (g0_v, g1_v)
        sems = (sem0, sem1)
        pend = pltpu.async_copy(g_h.at[0, pl.ds(base, RPW)], g0_v, sem0)
        pltpu.sync_copy(xw_h.at[pl.ds(base, RPW)], xw_v)
        pltpu.sync_copy(xwt_h.at[pl.ds(base, RPW)], xwt_v)
        pltpu.sync_copy(x_h.at[pl.ds(base, RPW)], x_v)
        pltpu.sync_copy(rad_h.at[wid], rad_v)
        rad = rad_v[...]
        lane = lax.iota(jnp.int32, L)
        _U = 2

        def _initb(c, carry):
            out = list(carry)
            for u in range(_U):
                for r in range(RPW):
                    sl = pl.ds((c * _U + u) * L, L)
                    lch = ((1.0 - 2.0 * x_v[r, sl])
                           * (xw_v[r, sl] + xwt_v[r, sl]) * 0.5)
                    l_v[r, sl] = lch
                    out[r] = jnp.maximum(out[r], jnp.abs(lch))
            return tuple(out)
        vmaxes = lax.fori_loop(0, NCHUNK // _U, _initb,
                               (jnp.zeros((L,), jnp.float32),) * RPW)
        m_rs = [jnp.max(vmaxes[r]) for r in range(RPW)]

        def _sumb(c, carry):
            out = list(carry)
            for u in range(_U):
                for r in range(RPW):
                    sl = pl.ds((c * _U + u) * L, L)
                    out[r] = out[r] + jnp.exp(l_v[r, sl] - m_rs[r])
            return tuple(out)
        accs = lax.fori_loop(0, NCHUNK // _U, _sumb,
                             (jnp.zeros((L,), jnp.float32),) * RPW)
        m_vec = jnp.zeros((L,), jnp.float32)
        s_vec = jnp.zeros((L,), jnp.float32)
        for r in range(RPW):
            lm = lane == r
            m_vec = jnp.where(lm, m_rs[r], m_vec)
            s_vec = jnp.where(lm, jnp.sum(accs[r]), s_vec)

        neg_inf = jnp.full((L,), -jnp.inf, jnp.float32)
        zero_i = jnp.zeros((L,), jnp.int32)
        for s in range(_MAXR):
            g_v = gbufs[s % 2]
            if s + 1 < _MAXR:
                nxt = pltpu.async_copy(
                    g_h.at[s + 1, pl.ds(base, RPW)],
                    gbufs[(s + 1) % 2], sems[(s + 1) % 2])
            pend.wait()
            pend = nxt if s + 1 < _MAXR else None
            flipm_all = jnp.broadcast_to(jnp.int32(s), (L,)) < rad
            s_pre = s_vec
            idx_all = jnp.zeros((L,), jnp.int32)
            sel_all = jnp.zeros((L,), jnp.float32)

            def _amax(c, carry, g_v=g_v):
                out = list(carry)
                for u in range(_U):
                    for r in range(RPW):
                        ci = c * _U + u
                        sl = pl.ds(ci * L, L)
                        t = l_v[r, sl] + g_v[r, sl]
                        vmax, vc = out[2 * r], out[2 * r + 1]
                        gt = t > vmax
                        out[2 * r] = jnp.where(gt, t, vmax)
                        out[2 * r + 1] = jnp.where(gt, ci, vc)
                return tuple(out)
            carry = lax.fori_loop(0, NCHUNK // _U, _amax,
                                  (neg_inf, zero_i) * RPW)
            for r in range(RPW):
                vmax, vc = carry[2 * r], carry[2 * r + 1]
                gmax = jnp.max(vmax)
                dcand = jnp.where(vmax == gmax, vc * L + lane, jnp.int32(2**30))
                idx_r = jnp.min(dcand)
                idxv = jnp.broadcast_to(idx_r, (L,))
                rv = jnp.broadcast_to(jnp.int32(r), (L,))
                selv = plsc.load_gather(l_v, [rv, idxv])
                lm = lane == r
                flip_r = flipm_all & lm
                idx_all = jnp.where(lm, idx_r, idx_all)
                sel_all = jnp.where(lm, selv, sel_all)
                plsc.store_scatter(l_v, [rv, idxv], -selv, mask=flip_r)
                bitv = plsc.load_gather(x_v, [rv, idxv])
                plsc.store_scatter(x_v, [rv, idxv], 1.0 - bitv, mask=flip_r)
                ds = jnp.exp(-selv - m_vec) - jnp.exp(selv - m_vec)
                s_vec = s_vec + jnp.where(flip_r, ds, 0.0)
            sall = jnp.broadcast_to(jnp.int32(s), (L,))
            plsc.store_scatter(idx_st, [lane, sall], idx_all)
            plsc.store_scatter(sel_st, [lane, sall], sel_all - m_vec)
            plsc.store_scatter(s_st, [lane, sall], s_pre)
        pltpu.sync_copy(x_v, y_h.at[pl.ds(base, RPW)])
        pltpu.sync_copy(idx_st.at[pl.ds(0, RPW)], idx_h.at[pl.ds(base, RPW)])
        pltpu.sync_copy(sel_st.at[pl.ds(0, RPW)], sel_h.at[pl.ds(base, RPW)])
        pltpu.sync_copy(s_st.at[pl.ds(0, RPW)], s_h.at[pl.ds(base, RPW)])

    return samp, NW, RPW, L


def _backward_kernel(yw_ref, ywt_ref, y_ref, x_ref, idx_ref, rm_ref,
                     sy_ref, sx_ref, self_ref, sf_ref, u_ref,
                     out_ref):
    lf = sx_ref[...] + jnp.sum(
        rm_ref[...] * (self_ref[...] - jnp.log(sf_ref[...])),
        axis=-1, keepdims=True)
    y = y_ref[...]
    delta = 1.0 - 2.0 * y
    grad = yw_ref[...] + ywt_ref[...]
    l = delta * grad * 0.5
    m = jnp.max(jnp.abs(l), axis=-1, keepdims=True)
    S = jnp.sum(jnp.exp(l - m), axis=-1, keepdims=True)
    iota = jax.lax.broadcasted_iota(jnp.int32, l.shape, 1)
    acc = jnp.zeros_like(m)
    for s in range(_MAXR - 1, -1, -1):
        idx = idx_ref[:, s:s + 1]
        onehot = iota == idx
        sel = jnp.sum(jnp.where(onehot, l, 0.0), axis=-1, keepdims=True)
        mask = rm_ref[:, s:s + 1]
        acc += mask * (sel - (m + jnp.log(S)))
        if s > 0:
            do = onehot & (mask > 0.0)
            l = jnp.where(do, -l, l)
            S = S + mask * (jnp.exp(-sel - m) - jnp.exp(sel - m))
    log_backwd = acc + sy_ref[...]
    log_acc = log_backwd - lf
    accept = (jnp.exp(log_acc) >= u_ref[...]).astype(jnp.float32)
    out_ref[...] = y * accept + (1.0 - accept) * x_ref[...]


def _backward(yw, ywt, y, x, idxarr, rmask, sy, sx, selfwd, sfwd, u):
    B, D = x.shape
    return pl.pallas_call(
        _backward_kernel,
        out_shape=jax.ShapeDtypeStruct((B, D), jnp.float32),
    )(yw, ywt, y, x, idxarr, rmask, sy, sx, selfwd, sfwd, u)


def kernel(x, W, b):
    B, D = x.shape
    key = jax.random.key(42)
    k_r, k_loop, k_acc = jax.random.split(key, 3)
    radius = jax.random.randint(k_r, (B, 1), 1, 2 * _R)
    r_mask = (jnp.arange(_MAXR)[None, :] < radius).astype(jnp.float32)
    G = jax.vmap(lambda s: jax.random.gumbel(
        jax.random.fold_in(k_loop, s), (B, D), jnp.float32))(jnp.arange(_MAXR))
    u = jax.random.uniform(k_acc, (B,)).reshape(B, 1)
    b2 = b.reshape(1, D)

    samp, NW, RPW, L = _sc_sampler_build(B, D)
    rad32 = jnp.zeros((NW, L), jnp.int32).at[:, :RPW].set(
        radius.reshape(NW, RPW))

    xw, xwt, sx = _interact(x, W, b2)
    y, idx_arr, selfwd, sfwd = samp(xw, xwt, x, G, rad32)
    yw, ywt, sy = _interact(y, W, b2)
    return _backward(yw, ywt, y, x, idx_arr, r_mask, sy, sx,
                     selfwd, sfwd, u)
